# baseline (speedup 1.0000x reference)
import functools
import jax
import jax.numpy as jnp
from jax import lax
from jax.experimental import pallas as pl
from jax.experimental.pallas import tpu as pltpu
from jax.experimental.pallas import tpu_sc as plsc

_N, _D, _K = 65536, 256, 1024
_BN = 512
_NB = _N // _BN

_NC, _NS = 2, 16
_NW = _NC * _NS
_RPW = _N // _NW
_C = 128
_NCHUNK = _RPW // _C


def _tc_body(x_ref, cb_ref, sim_ref, lab_ref):
    x = x_ref[...]
    cb = cb_ref[...]
    sim = lax.dot_general(x, cb, (((1,), (1,)), ((), ())),
                          preferred_element_type=jnp.float32)
    sim_ref[...] = sim
    lab_ref[0, 0, :] = jnp.argmax(sim, axis=1).astype(jnp.int32)


def _tc_call(input, codebook):
    return pl.pallas_call(
        _tc_body,
        grid=(_NB,),
        in_specs=[pl.BlockSpec((_BN, _D), lambda i: (i, 0)),
                  pl.BlockSpec((_K, _D), lambda i: (0, 0))],
        out_specs=[pl.BlockSpec((_BN, _K), lambda i: (i, 0)),
                   pl.BlockSpec((1, 1, _BN), lambda i: (i, 0, 0))],
        out_shape=[jax.ShapeDtypeStruct((_N, _K), jnp.float32),
                   jax.ShapeDtypeStruct((_NB, 1, _BN), jnp.int32)],
    )(input, codebook)


@functools.partial(
    pl.kernel,
    mesh=plsc.VectorSubcoreMesh(core_axis_name="c", subcore_axis_name="s"),
    out_type=jax.ShapeDtypeStruct((_N, _D), jnp.float32),
    scratch_types=[
        pltpu.VMEM((_C,), jnp.int32),
        pltpu.VMEM((_C, _D), jnp.float32),
        pltpu.SemaphoreType.DMA,
    ],
)
def _sc_gather(cb_hbm, lab_hbm, out_hbm, idx_v, rows_v, sem):
    wid = lax.axis_index("s") * _NC + lax.axis_index("c")
    w_base = wid * _RPW

    def body(i, carry):
        base = w_base + i * _C
        pltpu.sync_copy(lab_hbm.at[pl.ds(base, _C)], idx_v)
        pltpu.async_copy(cb_hbm.at[idx_v], rows_v, sem).wait()
        pltpu.sync_copy(rows_v, out_hbm.at[pl.ds(base, _C)])
        return carry

    lax.fori_loop(0, _NCHUNK, body, 0)


def kernel(input, codebook):
    sim, lab3 = _tc_call(input, codebook)
    labels = lab3.reshape(_N)
    preds = _sc_gather(codebook, labels)
    return (preds, labels.astype(jnp.int64), sim)


# speedup vs baseline: 1.0703x; 1.0703x over previous
import functools
import jax
import jax.numpy as jnp
from jax import lax
from jax.experimental import pallas as pl
from jax.experimental.pallas import tpu as pltpu
from jax.experimental.pallas import tpu_sc as plsc

_N, _D, _K = 65536, 256, 1024
_BN = 512
_NB = _N // _BN

_S = 4
_NCH = _N // _S
_NBC = _NCH // _BN

_NC, _NS = 2, 16
_NW = _NC * _NS
_RPWC = _NCH // _NW
_C = 128
_NLOOP = _RPWC // _C


def _tc_body0(x_ref, cb_ref, sim_ref, lab_ref):
    x = x_ref[...]
    cb = cb_ref[...]
    sim = lax.dot_general(x, cb, (((1,), (1,)), ((), ())),
                          preferred_element_type=jnp.float32)
    sim_ref[...] = sim
    lab_ref[0, 0, :] = jnp.argmax(sim, axis=1).astype(jnp.int32)


def _tc_body_alias(x_ref, cb_ref, simprev_ref, sim_ref, lab_ref):
    del simprev_ref
    _tc_body0(x_ref, cb_ref, sim_ref, lab_ref)


def _tc_chunk(ci, x, cb, sim_prev):
    i0 = ci * _NBC
    x_spec = pl.BlockSpec((_BN, _D), lambda j, i0=i0: (i0 + j, 0))
    cb_spec = pl.BlockSpec((_K, _D), lambda j: (0, 0))
    sim_spec = pl.BlockSpec((_BN, _K), lambda j, i0=i0: (i0 + j, 0))
    lab_spec = pl.BlockSpec((1, 1, _BN), lambda j: (j, 0, 0))
    out_shape = [jax.ShapeDtypeStruct((_N, _K), jnp.float32),
                 jax.ShapeDtypeStruct((_NBC, 1, _BN), jnp.int32)]
    if ci == 0:
        return pl.pallas_call(
            _tc_body0, grid=(_NBC,),
            in_specs=[x_spec, cb_spec],
            out_specs=[sim_spec, lab_spec],
            out_shape=out_shape,
        )(x, cb)
    return pl.pallas_call(
        _tc_body_alias, grid=(_NBC,),
        in_specs=[x_spec, cb_spec, pl.BlockSpec(memory_space=pl.ANY)],
        out_specs=[sim_spec, lab_spec],
        out_shape=out_shape,
        input_output_aliases={2: 0},
    )(x, cb, sim_prev)


_sc_mesh = plsc.VectorSubcoreMesh(core_axis_name="c", subcore_axis_name="s")


def _make_sc_gather(chunk_base):
    @functools.partial(
        pl.kernel,
        mesh=_sc_mesh,
        out_type=(),
        scratch_types=[
            pltpu.VMEM((_C,), jnp.int32),
            pltpu.VMEM((_C, _D), jnp.float32),
            pltpu.SemaphoreType.DMA,
        ],
    )
    def _sc_gather(cb_hbm, lab_hbm, out_hbm, idx_v, rows_v, sem):
        wid = lax.axis_index("s") * _NC + lax.axis_index("c")
        w_base = wid * _RPWC

        def body(i, carry):
            src = w_base + i * _C
            pltpu.sync_copy(lab_hbm.at[pl.ds(src, _C)], idx_v)
            pltpu.async_copy(cb_hbm.at[idx_v], rows_v, sem).wait()
            pltpu.sync_copy(rows_v, out_hbm.at[pl.ds(chunk_base + src, _C)])
            return carry

        lax.fori_loop(0, _NLOOP, body, 0)

    return _sc_gather


def _alloc_body(o_ref):
    pass


def kernel(input, codebook):
    preds_buf = pl.pallas_call(
        _alloc_body,
        out_specs=pl.BlockSpec(memory_space=pl.ANY),
        out_shape=jax.ShapeDtypeStruct((_N, _D), jnp.float32),
    )()
    preds_ref = jax.new_ref(preds_buf)

    sim = None
    lab_chunks = []
    for ci in range(_S):
        sim, lab3 = _tc_chunk(ci, input, codebook, sim)
        lab_chunk = lab3.reshape(_NCH)
        lab_chunks.append(lab_chunk)
        _make_sc_gather(ci * _NCH)(codebook, lab_chunk, preds_ref)

    labels = jnp.concatenate(lab_chunks)
    preds = preds_ref[...]
    return (preds, labels.astype(jnp.int64), sim)


# speedup vs baseline: 1.0734x; 1.0029x over previous
import functools
import jax
import jax.numpy as jnp
from jax import lax
from jax.experimental import pallas as pl
from jax.experimental.pallas import tpu as pltpu
from jax.experimental.pallas import tpu_sc as plsc

_N, _D, _K = 65536, 256, 1024
_BN = 512
_NB = _N // _BN

_S = 4
_NCH = _N // _S
_NBC = _NCH // _BN

_NC, _NS = 2, 16
_NW = _NC * _NS
_RPWC = _NCH // _NW
_C = 128
_NLOOP = _RPWC // _C


def _tc_body0(x_ref, cb_ref, sim_ref, lab_ref):
    x = x_ref[...]
    cb = cb_ref[...]
    sim = lax.dot_general(x, cb, (((1,), (1,)), ((), ())),
                          preferred_element_type=jnp.float32)
    sim_ref[...] = sim
    lab_ref[0, 0, :] = jnp.argmax(sim, axis=1).astype(jnp.int32)


def _tc_body_alias(x_ref, cb_ref, simprev_ref, sim_ref, lab_ref):
    del simprev_ref
    _tc_body0(x_ref, cb_ref, sim_ref, lab_ref)


def _tc_chunk(ci, x, cb, sim_prev):
    i0 = ci * _NBC
    x_spec = pl.BlockSpec((_BN, _D), lambda j, i0=i0: (i0 + j, 0))
    cb_spec = pl.BlockSpec((_K, _D), lambda j: (0, 0))
    sim_spec = pl.BlockSpec((_BN, _K), lambda j, i0=i0: (i0 + j, 0))
    lab_spec = pl.BlockSpec((1, 1, _BN), lambda j: (j, 0, 0))
    out_shape = [jax.ShapeDtypeStruct((_N, _K), jnp.float32),
                 jax.ShapeDtypeStruct((_NBC, 1, _BN), jnp.int32)]
    if ci == 0:
        return pl.pallas_call(
            _tc_body0, grid=(_NBC,),
            in_specs=[x_spec, cb_spec],
            out_specs=[sim_spec, lab_spec],
            out_shape=out_shape,
        )(x, cb)
    return pl.pallas_call(
        _tc_body_alias, grid=(_NBC,),
        in_specs=[x_spec, cb_spec, pl.BlockSpec(memory_space=pl.ANY)],
        out_specs=[sim_spec, lab_spec],
        out_shape=out_shape,
        input_output_aliases={2: 0},
    )(x, cb, sim_prev)


_sc_mesh = plsc.VectorSubcoreMesh(core_axis_name="c", subcore_axis_name="s")


_NBUF = 3
_KROWS_PER_SUB = _K // _NS


def _make_sc_gather(chunk_base):
    @functools.partial(
        pl.kernel,
        mesh=_sc_mesh,
        out_type=(),
        scratch_types=[
            pltpu.VMEM((_RPWC,), jnp.int32),
            [pltpu.VMEM((_C, _D), jnp.float32) for _ in range(_NBUF)],
            [pltpu.SemaphoreType.DMA for _ in range(_NBUF)],
            [pltpu.SemaphoreType.DMA for _ in range(_NBUF)],
        ],
    )
    def _sc_gather(cb_hbm, lab_hbm, out_hbm, idx_v, rows, gsem, wsem):
        cid = lax.axis_index("c")
        sid = lax.axis_index("s")
        wid = sid * _NC + cid
        w_base = wid * _RPWC

        pltpu.sync_copy(lab_hbm.at[pl.ds(w_base, _RPWC)], idx_v)

        def fire(r):
            return pltpu.async_copy(
                cb_hbm.at[idx_v.at[pl.ds(r * _C, _C)]], rows[r % _NBUF],
                gsem[r % _NBUF])

        gh = {r: fire(r) for r in range(min(_NBUF, _NLOOP))}
        wh = {}
        for r in range(_NLOOP):
            b = r % _NBUF
            gh[r].wait()
            wh[r] = pltpu.async_copy(
                rows[b], out_hbm.at[pl.ds(chunk_base + w_base + r * _C, _C)],
                wsem[b])
            nxt = r + _NBUF
            if nxt < _NLOOP:
                wh[r].wait()
                gh[nxt] = fire(nxt)
        for r in range(max(0, _NLOOP - _NBUF), _NLOOP):
            wh[r].wait()

    return _sc_gather


def _alloc_body(o_ref):
    pass


def kernel(input, codebook):
    preds_buf = pl.pallas_call(
        _alloc_body,
        out_specs=pl.BlockSpec(memory_space=pl.ANY),
        out_shape=jax.ShapeDtypeStruct((_N, _D), jnp.float32),
    )()
    preds_ref = jax.new_ref(preds_buf)

    sim = None
    lab_chunks = []
    for ci in range(_S):
        sim, lab3 = _tc_chunk(ci, input, codebook, sim)
        lab_chunk = lab3.reshape(_NCH)
        lab_chunks.append(lab_chunk)
        _make_sc_gather(ci * _NCH)(codebook, lab_chunk, preds_ref)

    labels = jnp.concatenate(lab_chunks)
    preds = preds_ref[...]
    return (preds, labels.astype(jnp.int64), sim)


# speedup vs baseline: 1.2167x; 1.1336x over previous
import functools
import jax
import jax.numpy as jnp
from jax import lax
from jax.experimental import pallas as pl
from jax.experimental.pallas import tpu as pltpu
from jax.experimental.pallas import tpu_sc as plsc

_N, _D, _K = 65536, 256, 1024
_BN = 512
_NB = _N // _BN

_S = 4
_NCH = _N // _S
_NBC = _NCH // _BN

_NC, _NS = 2, 16
_NW = _NC * _NS
_RPWC = _NCH // _NW
_C = 128
_NLOOP = _RPWC // _C
_KPS = _K // _NS


def _tc_body0(x_ref, cb_ref, sim_ref, lab_ref):
    x = x_ref[...]
    cb = cb_ref[...]
    sim = lax.dot_general(x, cb, (((1,), (1,)), ((), ())),
                          preferred_element_type=jnp.float32)
    sim_ref[...] = sim
    lab_ref[0, 0, :] = jnp.argmax(sim, axis=1).astype(jnp.int32)


def _tc_body_alias(x_ref, cb_ref, simprev_ref, sim_ref, lab_ref):
    del simprev_ref
    _tc_body0(x_ref, cb_ref, sim_ref, lab_ref)


def _tc_chunk(ci, x, cb, sim_prev):
    i0 = ci * _NBC
    x_spec = pl.BlockSpec((_BN, _D), lambda j, i0=i0: (i0 + j, 0))
    cb_spec = pl.BlockSpec((_K, _D), lambda j: (0, 0))
    sim_spec = pl.BlockSpec((_BN, _K), lambda j, i0=i0: (i0 + j, 0))
    lab_spec = pl.BlockSpec((1, 1, _BN), lambda j: (j, 0, 0))
    out_shape = [jax.ShapeDtypeStruct((_N, _K), jnp.float32),
                 jax.ShapeDtypeStruct((_NBC, 1, _BN), jnp.int32)]
    if ci == 0:
        return pl.pallas_call(
            _tc_body0, grid=(_NBC,),
            in_specs=[x_spec, cb_spec],
            out_specs=[sim_spec, lab_spec],
            out_shape=out_shape,
        )(x, cb)
    return pl.pallas_call(
        _tc_body_alias, grid=(_NBC,),
        in_specs=[x_spec, cb_spec, pl.BlockSpec(memory_space=pl.ANY)],
        out_specs=[sim_spec, lab_spec],
        out_shape=out_shape,
        input_output_aliases={2: 0},
    )(x, cb, sim_prev)


_sc_mesh = plsc.VectorSubcoreMesh(core_axis_name="c", subcore_axis_name="s")


def _make_sc_gather(chunk_base):
    @functools.partial(
        pl.kernel,
        mesh=_sc_mesh,
        out_type=(),
        scratch_types=[
            pltpu.VMEM((_RPWC,), jnp.int32),
            [pltpu.VMEM((_C, _D), jnp.float32) for _ in range(2)],
            pltpu.VMEM_SHARED((_K, _D), jnp.float32),
            [pltpu.SemaphoreType.DMA for _ in range(2)],
            [pltpu.SemaphoreType.DMA for _ in range(2)],
            pltpu.SemaphoreType.DMA,
        ],
    )
    def _sc_gather(cb_hbm, lab_hbm, out_hbm, idx_v, rows, cb_sh, rsem, wsem,
                   ssem):
        cid = lax.axis_index("c")
        sid = lax.axis_index("s")
        wid = sid * _NC + cid
        w_base = wid * _RPWC

        s0 = sid * _KPS
        sh = pltpu.async_copy(cb_hbm.at[pl.ds(s0, _KPS)],
                              cb_sh.at[pl.ds(s0, _KPS)], ssem)
        pltpu.sync_copy(lab_hbm.at[pl.ds(w_base, _RPWC)], idx_v)
        sh.wait()
        plsc.subcore_barrier()

        wh = {}
        for r in range(_NLOOP):
            b = r % 2
            if r >= 2:
                wh[r - 2].wait()

            def fire_group(g, carry):
                v = idx_v[pl.ds(r * _C + g * 16, 16)]
                for j in range(16):
                    pltpu.async_copy(cb_sh.at[pl.ds(v[j], 1)],
                                     rows[b].at[pl.ds(g * 16 + j, 1)],
                                     rsem[b])
                return carry

            lax.fori_loop(0, _C // 16, fire_group, 0)
            pltpu.make_async_copy(cb_hbm.at[pl.ds(0, _C)], rows[b],
                                  rsem[b]).wait()
            wh[r] = pltpu.async_copy(
                rows[b], out_hbm.at[pl.ds(chunk_base + w_base + r * _C, _C)],
                wsem[b])
        wh[_NLOOP - 2].wait()
        wh[_NLOOP - 1].wait()

    return _sc_gather


def _alloc_body(o_ref):
    pass


def kernel(input, codebook):
    preds_buf = pl.pallas_call(
        _alloc_body,
        out_specs=pl.BlockSpec(memory_space=pl.ANY),
        out_shape=jax.ShapeDtypeStruct((_N, _D), jnp.float32),
    )()
    preds_ref = jax.new_ref(preds_buf)

    sim = None
    lab_chunks = []
    for ci in range(_S):
        sim, lab3 = _tc_chunk(ci, input, codebook, sim)
        lab_chunk = lab3.reshape(_NCH)
        lab_chunks.append(lab_chunk)
        _make_sc_gather(ci * _NCH)(codebook, lab_chunk, preds_ref)

    labels = jnp.concatenate(lab_chunks)
    preds = preds_ref[...]
    return (preds, labels.astype(jnp.int64), sim)


# speedup vs baseline: 1.4815x; 1.2176x over previous
import functools
import jax
import jax.numpy as jnp
from jax import lax
from jax.experimental import pallas as pl
from jax.experimental.pallas import tpu as pltpu
from jax.experimental.pallas import tpu_sc as plsc

_N, _D, _K = 65536, 256, 1024
_BN = 1024
_NB = _N // _BN

_S = 4
_NCH = _N // _S
_NBC = _NCH // _BN

_NC, _NS = 2, 16
_NW = _NC * _NS
_RPWC = _NCH // _NW
_C = 128
_NLOOP = _RPWC // _C
_KPS = _K // _NS


def _tc_body0(x_ref, cb_ref, sim_ref, lab_ref):
    x = x_ref[...]
    cb = cb_ref[...]
    sim = lax.dot_general(x, cb, (((1,), (1,)), ((), ())),
                          preferred_element_type=jnp.float32)
    sim_ref[...] = sim
    lab_ref[0, 0, :] = jnp.argmax(sim, axis=1).astype(jnp.int32)


def _tc_body_alias(x_ref, cb_ref, simprev_ref, sim_ref, lab_ref):
    del simprev_ref
    _tc_body0(x_ref, cb_ref, sim_ref, lab_ref)


def _tc_chunk(ci, x, cb, sim_prev):
    i0 = ci * _NBC
    x_spec = pl.BlockSpec((_BN, _D), lambda j, i0=i0: (i0 + j, 0))
    cb_spec = pl.BlockSpec((_K, _D), lambda j: (0, 0))
    sim_spec = pl.BlockSpec((_BN, _K), lambda j, i0=i0: (i0 + j, 0))
    lab_spec = pl.BlockSpec((1, 1, _BN), lambda j: (j, 0, 0))
    out_shape = [jax.ShapeDtypeStruct((_N, _K), jnp.float32),
                 jax.ShapeDtypeStruct((_NBC, 1, _BN), jnp.int32)]
    if ci == 0:
        return pl.pallas_call(
            _tc_body0, grid=(_NBC,),
            in_specs=[x_spec, cb_spec],
            out_specs=[sim_spec, lab_spec],
            out_shape=out_shape,
        )(x, cb)
    return pl.pallas_call(
        _tc_body_alias, grid=(_NBC,),
        in_specs=[x_spec, cb_spec, pl.BlockSpec(memory_space=pl.ANY)],
        out_specs=[sim_spec, lab_spec],
        out_shape=out_shape,
        input_output_aliases={2: 0},
    )(x, cb, sim_prev)


_sc_mesh = plsc.VectorSubcoreMesh(core_axis_name="c", subcore_axis_name="s")


def _make_sc_gather(chunk_base):
    @functools.partial(
        pl.kernel,
        mesh=_sc_mesh,
        out_type=(),
        scratch_types=[
            pltpu.VMEM((_RPWC,), jnp.int32),
            [pltpu.VMEM((_C, _D), jnp.float32) for _ in range(2)],
            pltpu.VMEM_SHARED((_K, _D), jnp.float32),
            [pltpu.SemaphoreType.DMA for _ in range(2)],
            [pltpu.SemaphoreType.DMA for _ in range(2)],
            pltpu.SemaphoreType.DMA,
        ],
    )
    def _sc_gather(cb_hbm, lab_hbm, out_hbm, idx_v, rows, cb_sh, rsem, wsem,
                   ssem):
        cid = lax.axis_index("c")
        sid = lax.axis_index("s")
        wid = sid * _NC + cid
        w_base = wid * _RPWC

        s0 = sid * _KPS
        sh = pltpu.async_copy(cb_hbm.at[pl.ds(s0, _KPS)],
                              cb_sh.at[pl.ds(s0, _KPS)], ssem)
        pltpu.sync_copy(lab_hbm.at[pl.ds(w_base, _RPWC)], idx_v)
        sh.wait()
        plsc.subcore_barrier()

        wh = {}
        for r in range(_NLOOP):
            b = r % 2
            if r >= 2:
                wh[r - 2].wait()

            def fire_group(g, carry):
                v = idx_v[pl.ds(r * _C + g * 16, 16)]
                for j in range(16):
                    pltpu.async_copy(cb_sh.at[pl.ds(v[j], 1)],
                                     rows[b].at[pl.ds(g * 16 + j, 1)],
                                     rsem[b])
                return carry

            lax.fori_loop(0, _C // 16, fire_group, 0)
            pltpu.make_async_copy(cb_hbm.at[pl.ds(0, _C)], rows[b],
                                  rsem[b]).wait()
            wh[r] = pltpu.async_copy(
                rows[b], out_hbm.at[pl.ds(chunk_base + w_base + r * _C, _C)],
                wsem[b])
        wh[_NLOOP - 2].wait()
        wh[_NLOOP - 1].wait()

    return _sc_gather


def _alloc_body(o_ref):
    pass


def kernel(input, codebook):
    preds_buf = pl.pallas_call(
        _alloc_body,
        out_specs=pl.BlockSpec(memory_space=pl.ANY),
        out_shape=jax.ShapeDtypeStruct((_N, _D), jnp.float32),
    )()
    preds_ref = jax.new_ref(preds_buf)

    sim = None
    lab_chunks = []
    for ci in range(_S):
        sim, lab3 = _tc_chunk(ci, input, codebook, sim)
        lab_chunk = lab3.reshape(_NCH)
        lab_chunks.append(lab_chunk)
        _make_sc_gather(ci * _NCH)(codebook, lab_chunk, preds_ref)

    labels = jnp.concatenate(lab_chunks)
    preds = preds_ref[...]
    return (preds, labels.astype(jnp.int64), sim)
